# baseline (speedup 1.0000x reference)
import functools

import jax
import jax.numpy as jnp
from jax import lax
from jax.experimental import pallas as pl
from jax.experimental.pallas import tpu as pltpu
from jax.experimental.pallas import tpu_sc as plsc

N_NODES = 10000
N_EDGES = 320000
D = 128
E_IN = 48
TIME_DIM = 32
EPSILON = 0.1
GAMMA = 0.1

NC, NS = 2, 16
NW = NC * NS
E_PAD = 327680
E_PER_W = E_PAD // NW
N_ACC = 10240
TRASH = N_NODES
INV_SQRT_D = 1.0 / (128.0 ** 0.5)

LCH = 1024
N_LCH = E_PER_W // LCH

CHA = 80
N_CHA = E_PER_W // CHA
N_PAIR_A = N_CHA // 2
CH = 64
N_CH = E_PER_W // CH
N_PAIR = N_CH // 2


def _make_lu_gather():
    mesh = plsc.VectorSubcoreMesh(core_axis_name="c", subcore_axis_name="s")

    @functools.partial(
        pl.kernel,
        mesh=mesh,
        compiler_params=pltpu.CompilerParams(needs_layout_passes=False),
        out_type=jax.ShapeDtypeStruct((E_PAD,), jnp.float32),
        scratch_types=[
            pltpu.VMEM((N_NODES,), jnp.float32),
            pltpu.VMEM((N_LCH, LCH), jnp.int32),
            pltpu.VMEM((N_LCH, LCH), jnp.float32),
            pltpu.SemaphoreType.DMA,
            pltpu.SemaphoreType.DMA,
        ],
    )
    def lu_k(lu_h, src_h, lug_h, lutab, srcb, outb, isem, wsem):
        wid = lax.axis_index("s") * NC + lax.axis_index("c")
        wbase = wid * E_PER_W
        pltpu.sync_copy(lu_h, lutab)
        for i in range(N_LCH):
            pltpu.async_copy(src_h.at[pl.ds(wbase + i * LCH, LCH)],
                             srcb.at[i], isem)
        for i in range(N_LCH):
            pltpu.make_async_copy(src_h.at[pl.ds(wbase, LCH)],
                                  srcb.at[i], isem).wait()
            for g in range(LCH // 16):
                idx16 = srcb[i, pl.ds(g * 16, 16)]
                outb[i, pl.ds(g * 16, 16)] = plsc.load_gather(lutab, [idx16])
            pltpu.async_copy(outb.at[i],
                             lug_h.at[pl.ds(wbase + i * LCH, LCH)], wsem)
        for i in range(N_LCH):
            pltpu.make_async_copy(outb.at[i],
                                  lug_h.at[pl.ds(wbase, LCH)], wsem).wait()

    return lu_k


_lu_gather = _make_lu_gather()


def _node_proj_kernel(z, wq, wk, wv, ws, bq, bk, bv, bs, q_o, k_o, v_o, s_o):
    zb = z[...]
    q_o[...] = jnp.dot(zb, wq[...], preferred_element_type=jnp.float32) + bq[...]
    k_o[...] = jnp.dot(zb, wk[...], preferred_element_type=jnp.float32) + bk[...]
    v_o[...] = jnp.dot(zb, wv[...], preferred_element_type=jnp.float32) + bv[...]
    s_o[...] = jnp.dot(zb, ws[...], preferred_element_type=jnp.float32) + bs[...]


def _node_proj(z, wqT, wkT, wvT, wsT, bq, bk, bv, bs):
    n = z.shape[0]
    blk = 1000
    full = lambda r, c: pl.BlockSpec((r, c), lambda i: (0, 0))
    row = pl.BlockSpec((blk, D), lambda i: (i, 0))
    return pl.pallas_call(
        _node_proj_kernel,
        grid=(n // blk,),
        in_specs=[row, full(D, D), full(D, D), full(D, D), full(D, D),
                  full(1, D), full(1, D), full(1, D), full(1, D)],
        out_specs=[row, row, row, row],
        out_shape=[jax.ShapeDtypeStruct((n, D), jnp.float32)] * 4,
    )(z, wqT, wkT, wvT, wsT, bq, bk, bv, bs)


def _edge_feat_kernel(lug, t, msg, twT, tb, weT, e_o):
    rel = lug[...] - t[...]
    enc = jnp.cos(rel * twT[...] + tb[...])
    we = weT[...]
    ef = (jnp.dot(enc, we[0:TIME_DIM], preferred_element_type=jnp.float32)
          + jnp.dot(msg[...], we[TIME_DIM:E_IN],
                    preferred_element_type=jnp.float32)).astype(jnp.bfloat16)
    bits = jax.lax.bitcast_convert_type(ef, jnp.int16)
    lo = bits[:, 0:D // 2].astype(jnp.int32) & jnp.int32(0xFFFF)
    hi = bits[:, D // 2:D].astype(jnp.int32) << 16
    e_o[...] = lo | hi


def _edge_feat(lug2, t2, msg, twT, tb, weT):
    blk = 2048
    return pl.pallas_call(
        _edge_feat_kernel,
        grid=(E_PAD // blk,),
        in_specs=[
            pl.BlockSpec((blk, 1), lambda i: (i, 0)),
            pl.BlockSpec((blk, 1), lambda i: (i, 0)),
            pl.BlockSpec((blk, 16), lambda i: (i, 0)),
            pl.BlockSpec((1, TIME_DIM), lambda i: (0, 0)),
            pl.BlockSpec((1, TIME_DIM), lambda i: (0, 0)),
            pl.BlockSpec((E_IN, D), lambda i: (0, 0)),
        ],
        out_specs=pl.BlockSpec((blk, D // 2), lambda i: (i, 0)),
        out_shape=jax.ShapeDtypeStruct((E_PAD, D // 2), jnp.int32),
    )(lug2, t2, msg, twT, tb, weT)


def _make_edge_pass():
    mesh = plsc.VectorSubcoreMesh(core_axis_name="c", subcore_axis_name="s")

    @functools.partial(
        pl.kernel,
        mesh=mesh,
        compiler_params=pltpu.CompilerParams(needs_layout_passes=False),
        out_type=[
            jax.ShapeDtypeStruct((E_PAD, D), jnp.float32),
            jax.ShapeDtypeStruct((E_PAD,), jnp.float32),
        ],
        scratch_types=[
            pltpu.VMEM((E_PER_W,), jnp.int32),
            pltpu.VMEM((E_PER_W,), jnp.int32),
            pltpu.VMEM((2, CHA, D), jnp.float32),
            pltpu.VMEM((2, CHA, D), jnp.float32),
            pltpu.VMEM((2, CHA, D), jnp.float32),
            pltpu.VMEM((2, CHA, D // 2), jnp.int32),
            pltpu.VMEM((2, CHA, D), jnp.float32),
            pltpu.VMEM((2, CHA + 16), jnp.float32),
            pltpu.SemaphoreType.DMA,
            pltpu.SemaphoreType.DMA,
            pltpu.SemaphoreType.DMA,
            pltpu.SemaphoreType.DMA,
        ],
    )
    def edge_k(q_h, k_h, v_h, e_h, src_h, dst_h,
               wv_h, w_h,
               srcb, dstb, qb, kb, vb, eb, wvb, wb,
               gsem0, gsem1, wsem0, wsem1):
        wid = lax.axis_index("s") * NC + lax.axis_index("c")
        wbase = wid * E_PER_W
        gsems = (gsem0, gsem1)
        wsems = (wsem0, wsem1)

        pltpu.sync_copy(src_h.at[pl.ds(wbase, E_PER_W)], srcb)
        pltpu.sync_copy(dst_h.at[pl.ds(wbase, E_PER_W)], dstb)

        def fire(i, p):
            base = wbase + i * CHA
            so = srcb.at[pl.ds(i * CHA, CHA)]
            do = dstb.at[pl.ds(i * CHA, CHA)]
            pltpu.async_copy(q_h.at[do], qb.at[p], gsems[p])
            pltpu.async_copy(k_h.at[so], kb.at[p], gsems[p])
            pltpu.async_copy(v_h.at[so], vb.at[p], gsems[p])
            pltpu.async_copy(e_h.at[pl.ds(base, CHA)], eb.at[p], gsems[p])

        def wait_gathers(p):
            so = srcb.at[pl.ds(0, CHA)]
            do = dstb.at[pl.ds(0, CHA)]
            pltpu.make_async_copy(q_h.at[do], qb.at[p], gsems[p]).wait()
            pltpu.make_async_copy(k_h.at[so], kb.at[p], gsems[p]).wait()
            pltpu.make_async_copy(v_h.at[so], vb.at[p], gsems[p]).wait()
            pltpu.make_async_copy(e_h.at[pl.ds(wbase, CHA)], eb.at[p],
                                  gsems[p]).wait()

        def fire_writes(i, p):
            base = wbase + i * CHA
            pltpu.async_copy(wvb.at[p], wv_h.at[pl.ds(base, CHA)], wsems[p])
            pltpu.async_copy(wb.at[p, pl.ds(0, CHA)],
                             w_h.at[pl.ds(base, CHA)], wsems[p])

        def wait_writes(p):
            pltpu.make_async_copy(wvb.at[p], wv_h.at[pl.ds(wbase, CHA)],
                                  wsems[p]).wait()
            pltpu.make_async_copy(wb.at[p, pl.ds(0, CHA)],
                                  w_h.at[pl.ds(wbase, CHA)], wsems[p]).wait()

        def compute(p):
            lanes = lax.broadcasted_iota(jnp.int32, (16,), 0)
            for g in range(CHA // 16):
                def gbody(ii, vec):
                    for j in range(4):
                        e = g * 16 + ii * 4 + j
                        acc = jnp.zeros((16,), jnp.float32)
                        for u in range(4):
                            x = eb[p, e, pl.ds(u * 16, 16)]
                            ea = plsc.bitcast(x << 16, jnp.float32)
                            eb2 = plsc.bitcast(
                                x & jnp.int32(-65536), jnp.float32)
                            sa = pl.ds(u * 32, 16)
                            sb = pl.ds(u * 32 + 16, 16)
                            acc = acc + qb[p, e, sa] * (kb[p, e, sa] + ea)
                            acc = acc + qb[p, e, sb] * (kb[p, e, sb] + eb2)
                        vec = jnp.where(lanes == ii * 4 + j,
                                        jnp.sum(acc), vec)
                    return vec

                vec = lax.fori_loop(0, 4, gbody,
                                    jnp.zeros((16,), jnp.float32))
                wb[p, pl.ds(g * 16, 16)] = jnp.exp(vec * INV_SQRT_D)

            def vbody(ii, carry):
                for j in range(4):
                    e = ii * 4 + j
                    w = wb[p, pl.ds(e, 16)][0]
                    for u in range(4):
                        x = eb[p, e, pl.ds(u * 16, 16)]
                        ea = plsc.bitcast(x << 16, jnp.float32)
                        eb2 = plsc.bitcast(x & jnp.int32(-65536), jnp.float32)
                        sa = pl.ds(u * 32, 16)
                        sb = pl.ds(u * 32 + 16, 16)
                        wvb[p, e, sa] = w * (vb[p, e, sa] + ea)
                        wvb[p, e, sb] = w * (vb[p, e, sb] + eb2)
                return carry

            lax.fori_loop(0, CHA // 4, vbody, 0)

        fire(0, 0)

        def pair(g, carry):
            i0 = 2 * g

            @pl.when(g > 0)
            def _():
                wait_writes(1)
            fire(i0 + 1, 1)
            wait_gathers(0)

            @pl.when(g > 0)
            def _():
                wait_writes(0)
            compute(0)
            fire_writes(i0, 0)

            @pl.when(g < N_PAIR_A - 1)
            def _():
                fire(i0 + 2, 0)
            wait_gathers(1)
            compute(1)
            fire_writes(i0 + 1, 1)
            return carry

        lax.fori_loop(0, N_PAIR_A, pair, 0)
        wait_writes(0)
        wait_writes(1)

    return edge_k


_edge_pass = _make_edge_pass()


def _make_scatter_kernel():
    mesh = plsc.VectorSubcoreMesh(core_axis_name="c", subcore_axis_name="s")
    rows_per_tile = N_ACC // NS

    @functools.partial(
        pl.kernel,
        mesh=mesh,
        compiler_params=pltpu.CompilerParams(needs_layout_passes=False),
        out_type=[
            jax.ShapeDtypeStruct((NC, N_ACC, D), jnp.float32),
            jax.ShapeDtypeStruct((NW, N_ACC), jnp.float32),
        ],
        scratch_types=[
            pltpu.VMEM((N_CH, CH), jnp.int32),
            pltpu.VMEM((2, CH, D), jnp.float32),
            pltpu.VMEM((2, CH), jnp.float32),
            pltpu.VMEM((N_ACC,), jnp.float32),
            pltpu.VMEM_SHARED((N_ACC, D), jnp.float32),
            pltpu.SemaphoreType.DMA,
            pltpu.SemaphoreType.DMA,
        ],
    )
    def scatter_k(wv_h, w_h, dst2_h, z128_h, zden_h,
                  acc128_o, den_o,
                  dstb, wvb, wb, den_v, acc_sh, isem0, isem1):
        cid = lax.axis_index("c")
        sid = lax.axis_index("s")
        wid = sid * NC + cid
        wbase = wid * E_PER_W
        r0 = sid * rows_per_tile
        isems = (isem0, isem1)

        pltpu.sync_copy(z128_h.at[pl.ds(r0, rows_per_tile)],
                        acc_sh.at[pl.ds(r0, rows_per_tile)])
        pltpu.sync_copy(zden_h, den_v)
        pltpu.sync_copy(dst2_h.at[pl.ds(wid * N_CH, N_CH)], dstb)
        plsc.subcore_barrier()

        def fire(i, p):
            base = wbase + i * CH
            pltpu.async_copy(wv_h.at[pl.ds(base, CH)], wvb.at[p], isems[p])
            pltpu.async_copy(w_h.at[pl.ds(base, CH)], wb.at[p], isems[p])

        def wait_in(p):
            pltpu.make_async_copy(wv_h.at[pl.ds(wbase, CH)], wvb.at[p],
                                  isems[p]).wait()
            pltpu.make_async_copy(w_h.at[pl.ds(wbase, CH)], wb.at[p],
                                  isems[p]).wait()

        def consume(i, p):
            pltpu.sync_copy(wvb.at[p], acc_sh.at[dstb.at[i]], add=True)
            for g in range(CH // 16):
                s = pl.ds(g * 16, 16)
                plsc.addupdate_scatter(den_v, [dstb[i, s]], wb[p, s])

        fire(0, 0)

        def pair(g, carry):
            i0 = 2 * g
            fire(i0 + 1, 1)
            wait_in(0)
            consume(i0, 0)

            @pl.when(g < N_PAIR - 1)
            def _():
                fire(i0 + 2, 0)
            wait_in(1)
            consume(i0 + 1, 1)
            return carry

        lax.fori_loop(0, N_PAIR, pair, 0)
        plsc.subcore_barrier()

        pltpu.sync_copy(acc_sh.at[pl.ds(r0, rows_per_tile)],
                        acc128_o.at[cid, pl.ds(r0, rows_per_tile)])
        pltpu.sync_copy(den_v, den_o.at[wid])

    return scatter_k


_scatter = _make_scatter_kernel()


def _final_kernel(a, dp, z, zskip, out):
    denom = jnp.sum(dp[...], axis=1, keepdims=True) + 1e-16
    agg = a[0] + a[1]
    c = agg / denom + zskip[...]
    zb = z[...]
    out[...] = zb + EPSILON * jnp.tanh(c - GAMMA * zb)


def _final(acc128, den_p, z, zskip):
    blk = 1000
    return pl.pallas_call(
        _final_kernel,
        grid=(N_NODES // blk,),
        in_specs=[
            pl.BlockSpec((NC, blk, D), lambda i: (0, i, 0)),
            pl.BlockSpec((blk, NW), lambda i: (i, 0)),
            pl.BlockSpec((blk, D), lambda i: (i, 0)),
            pl.BlockSpec((blk, D), lambda i: (i, 0)),
        ],
        out_specs=pl.BlockSpec((blk, D), lambda i: (i, 0)),
        out_shape=jax.ShapeDtypeStruct((N_NODES, D), jnp.float32),
    )(acc128[:, :N_NODES], den_p.T[:N_NODES], z, zskip)


def kernel(x, n_id, msg, t, edge_index, memory, last_update, time_w, time_b,
           Wq, bq, Wk, bk, Wv, bv, We, Ws, bs):
    z = memory
    lu = last_update

    src = edge_index[0]
    dst = edge_index[1]
    pad = E_PAD - N_EDGES
    spread = jnp.arange(pad, dtype=jnp.int32)
    src_p = jnp.concatenate([src, spread % N_NODES])
    dstg_p = jnp.concatenate([dst, spread % N_NODES])
    dsts_p = jnp.concatenate([dst, TRASH + spread % (N_ACC - N_NODES)])
    t_p = jnp.concatenate([t, jnp.zeros((pad,), jnp.float32)])
    msg_p = jnp.concatenate([msg, jnp.zeros((pad, 16), jnp.float32)])

    lug = _lu_gather(lu, src_p)

    q, k, v, zskip = _node_proj(
        z, Wq.T, Wk.T, Wv.T, Ws.T,
        bq.reshape(1, D), bk.reshape(1, D), bv.reshape(1, D), bs.reshape(1, D))
    u4 = jnp.arange(4, dtype=jnp.int32)[:, None] * 32
    l16 = jnp.arange(16, dtype=jnp.int32)[None, :]
    lo_order = (u4 + l16).reshape(-1)
    order = jnp.concatenate([lo_order, lo_order + 16])
    E = _edge_feat(lug.reshape(E_PAD, 1), t_p.reshape(E_PAD, 1), msg_p,
                   time_w.reshape(1, TIME_DIM), time_b.reshape(1, TIME_DIM),
                   We.T[:, order])

    wv, w = _edge_pass(q, k, v, E, src_p, dstg_p)

    zeros128 = jnp.zeros((N_ACC, D), jnp.float32)
    zerosden = jnp.zeros((N_ACC,), jnp.float32)
    acc128, den_p = _scatter(wv, w, dsts_p.reshape(NW * N_CH, CH),
                             zeros128, zerosden)

    return _final(acc128, den_p, z, zskip)


# speedup vs baseline: 1.0425x; 1.0425x over previous
import functools

import jax
import jax.numpy as jnp
from jax import lax
from jax.experimental import pallas as pl
from jax.experimental.pallas import tpu as pltpu
from jax.experimental.pallas import tpu_sc as plsc

N_NODES = 10000
N_EDGES = 320000
D = 128
E_IN = 48
TIME_DIM = 32
EPSILON = 0.1
GAMMA = 0.1

NC, NS = 2, 16
NW = NC * NS
E_PAD = 327680
E_PER_W = E_PAD // NW
N_ACC = 10240
TRASH = N_NODES
INV_SQRT_D = 1.0 / (128.0 ** 0.5)

LCH = 1024
N_LCH = E_PER_W // LCH

CH = 64
N_CH = E_PER_W // CH
N_PAIR = N_CH // 2


def _make_lu_gather():
    mesh = plsc.VectorSubcoreMesh(core_axis_name="c", subcore_axis_name="s")

    @functools.partial(
        pl.kernel,
        mesh=mesh,
        compiler_params=pltpu.CompilerParams(needs_layout_passes=False),
        out_type=jax.ShapeDtypeStruct((E_PAD,), jnp.float32),
        scratch_types=[
            pltpu.VMEM((N_NODES,), jnp.float32),
            pltpu.VMEM((N_LCH, LCH), jnp.int32),
            pltpu.VMEM((N_LCH, LCH), jnp.float32),
            pltpu.SemaphoreType.DMA,
            pltpu.SemaphoreType.DMA,
        ],
    )
    def lu_k(lu_h, src_h, lug_h, lutab, srcb, outb, isem, wsem):
        wid = lax.axis_index("s") * NC + lax.axis_index("c")
        wbase = wid * E_PER_W
        pltpu.sync_copy(lu_h, lutab)
        for i in range(N_LCH):
            pltpu.async_copy(src_h.at[pl.ds(wbase + i * LCH, LCH)],
                             srcb.at[i], isem)
        for i in range(N_LCH):
            pltpu.make_async_copy(src_h.at[pl.ds(wbase, LCH)],
                                  srcb.at[i], isem).wait()
            for g in range(LCH // 16):
                idx16 = srcb[i, pl.ds(g * 16, 16)]
                outb[i, pl.ds(g * 16, 16)] = plsc.load_gather(lutab, [idx16])
            pltpu.async_copy(outb.at[i],
                             lug_h.at[pl.ds(wbase + i * LCH, LCH)], wsem)
        for i in range(N_LCH):
            pltpu.make_async_copy(outb.at[i],
                                  lug_h.at[pl.ds(wbase, LCH)], wsem).wait()

    return lu_k


_lu_gather = _make_lu_gather()


def _pack_cols(x):
    bits = jax.lax.bitcast_convert_type(x.astype(jnp.bfloat16), jnp.int16)
    lo = bits[:, 0:D // 2].astype(jnp.int32) & jnp.int32(0xFFFF)
    hi = bits[:, D // 2:D].astype(jnp.int32) << 16
    return lo | hi


def _node_proj_kernel(z, wq, wk, wv, ws, bq, bk, bv, bs, q_o, kv_o, s_o):
    zb = z[...]
    q_o[...] = jnp.dot(zb, wq[...], preferred_element_type=jnp.float32) + bq[...]
    k = jnp.dot(zb, wk[...], preferred_element_type=jnp.float32) + bk[...]
    v = jnp.dot(zb, wv[...], preferred_element_type=jnp.float32) + bv[...]
    kv_o[:, 0:D // 2] = _pack_cols(k)
    kv_o[:, D // 2:D] = _pack_cols(v)
    s_o[...] = jnp.dot(zb, ws[...], preferred_element_type=jnp.float32) + bs[...]


def _node_proj(z, wqT, wkT, wvT, wsT, bq, bk, bv, bs):
    n = z.shape[0]
    blk = 1000
    full = lambda r, c: pl.BlockSpec((r, c), lambda i: (0, 0))
    row = pl.BlockSpec((blk, D), lambda i: (i, 0))
    return pl.pallas_call(
        _node_proj_kernel,
        grid=(n // blk,),
        in_specs=[row, full(D, D), full(D, D), full(D, D), full(D, D),
                  full(1, D), full(1, D), full(1, D), full(1, D)],
        out_specs=[row, row, row],
        out_shape=[jax.ShapeDtypeStruct((n, D), jnp.float32),
                   jax.ShapeDtypeStruct((n, D), jnp.int32),
                   jax.ShapeDtypeStruct((n, D), jnp.float32)],
    )(z, wqT, wkT, wvT, wsT, bq, bk, bv, bs)


def _edge_feat_kernel(lug, t, msg, twT, tb, weT, e_o):
    rel = lug[...] - t[...]
    enc = jnp.cos(rel * twT[...] + tb[...])
    we = weT[...]
    ef = (jnp.dot(enc, we[0:TIME_DIM], preferred_element_type=jnp.float32)
          + jnp.dot(msg[...], we[TIME_DIM:E_IN],
                    preferred_element_type=jnp.float32)).astype(jnp.bfloat16)
    bits = jax.lax.bitcast_convert_type(ef, jnp.int16)
    lo = bits[:, 0:D // 2].astype(jnp.int32) & jnp.int32(0xFFFF)
    hi = bits[:, D // 2:D].astype(jnp.int32) << 16
    e_o[...] = lo | hi


def _edge_feat(lug2, t2, msg, twT, tb, weT):
    blk = 2048
    return pl.pallas_call(
        _edge_feat_kernel,
        grid=(E_PAD // blk,),
        in_specs=[
            pl.BlockSpec((blk, 1), lambda i: (i, 0)),
            pl.BlockSpec((blk, 1), lambda i: (i, 0)),
            pl.BlockSpec((blk, 16), lambda i: (i, 0)),
            pl.BlockSpec((1, TIME_DIM), lambda i: (0, 0)),
            pl.BlockSpec((1, TIME_DIM), lambda i: (0, 0)),
            pl.BlockSpec((E_IN, D), lambda i: (0, 0)),
        ],
        out_specs=pl.BlockSpec((blk, D // 2), lambda i: (i, 0)),
        out_shape=jax.ShapeDtypeStruct((E_PAD, D // 2), jnp.int32),
    )(lug2, t2, msg, twT, tb, weT)


def _make_edge_pass():
    mesh = plsc.VectorSubcoreMesh(core_axis_name="c", subcore_axis_name="s")

    @functools.partial(
        pl.kernel,
        mesh=mesh,
        compiler_params=pltpu.CompilerParams(needs_layout_passes=False),
        out_type=[
            jax.ShapeDtypeStruct((E_PAD, D), jnp.float32),
            jax.ShapeDtypeStruct((E_PAD,), jnp.float32),
        ],
        scratch_types=[
            pltpu.VMEM((E_PER_W,), jnp.int32),
            pltpu.VMEM((E_PER_W,), jnp.int32),
            pltpu.VMEM((2, CH, D), jnp.float32),
            pltpu.VMEM((2, CH, D), jnp.int32),
            pltpu.VMEM((2, CH, D // 2), jnp.int32),
            pltpu.VMEM((2, CH, D), jnp.float32),
            pltpu.VMEM((2, CH + 16), jnp.float32),
            pltpu.SemaphoreType.DMA,
            pltpu.SemaphoreType.DMA,
            pltpu.SemaphoreType.DMA,
            pltpu.SemaphoreType.DMA,
        ],
    )
    def edge_k(q_h, kv_h, e_h, src_h, dst_h,
               wv_h, w_h,
               srcb, dstb, qb, kvb, eb, wvb, wb,
               gsem0, gsem1, wsem0, wsem1):
        wid = lax.axis_index("s") * NC + lax.axis_index("c")
        wbase = wid * E_PER_W
        gsems = (gsem0, gsem1)
        wsems = (wsem0, wsem1)

        pltpu.sync_copy(src_h.at[pl.ds(wbase, E_PER_W)], srcb)
        pltpu.sync_copy(dst_h.at[pl.ds(wbase, E_PER_W)], dstb)

        def fire(i, p):
            base = wbase + i * CH
            so = srcb.at[pl.ds(i * CH, CH)]
            do = dstb.at[pl.ds(i * CH, CH)]
            pltpu.async_copy(q_h.at[do], qb.at[p], gsems[p])
            pltpu.async_copy(kv_h.at[so], kvb.at[p], gsems[p])
            pltpu.async_copy(e_h.at[pl.ds(base, CH)], eb.at[p], gsems[p])

        def wait_gathers(p):
            so = srcb.at[pl.ds(0, CH)]
            do = dstb.at[pl.ds(0, CH)]
            pltpu.make_async_copy(q_h.at[do], qb.at[p], gsems[p]).wait()
            pltpu.make_async_copy(kv_h.at[so], kvb.at[p], gsems[p]).wait()
            pltpu.make_async_copy(e_h.at[pl.ds(wbase, CH)], eb.at[p],
                                  gsems[p]).wait()

        def fire_writes(i, p):
            base = wbase + i * CH
            pltpu.async_copy(wvb.at[p], wv_h.at[pl.ds(base, CH)], wsems[p])
            pltpu.async_copy(wb.at[p, pl.ds(0, CH)],
                             w_h.at[pl.ds(base, CH)], wsems[p])

        def wait_writes(p):
            pltpu.make_async_copy(wvb.at[p], wv_h.at[pl.ds(wbase, CH)],
                                  wsems[p]).wait()
            pltpu.make_async_copy(wb.at[p, pl.ds(0, CH)],
                                  w_h.at[pl.ds(wbase, CH)], wsems[p]).wait()

        def compute(p):
            lanes = lax.broadcasted_iota(jnp.int32, (16,), 0)
            for g in range(CH // 16):
                def gbody(ii, vec):
                    for j in range(4):
                        e = g * 16 + ii * 4 + j
                        acc = jnp.zeros((16,), jnp.float32)
                        for u in range(4):
                            x = eb[p, e, pl.ds(u * 16, 16)]
                            ea = plsc.bitcast(x << 16, jnp.float32)
                            eb2 = plsc.bitcast(
                                x & jnp.int32(-65536), jnp.float32)
                            xk = kvb[p, e, pl.ds(u * 16, 16)]
                            ka = plsc.bitcast(xk << 16, jnp.float32)
                            kb2 = plsc.bitcast(
                                xk & jnp.int32(-65536), jnp.float32)
                            sa = pl.ds(u * 32, 16)
                            sb = pl.ds(u * 32 + 16, 16)
                            acc = acc + qb[p, e, sa] * (ka + ea)
                            acc = acc + qb[p, e, sb] * (kb2 + eb2)
                        vec = jnp.where(lanes == ii * 4 + j,
                                        jnp.sum(acc), vec)
                    return vec

                vec = lax.fori_loop(0, 4, gbody,
                                    jnp.zeros((16,), jnp.float32))
                wb[p, pl.ds(g * 16, 16)] = jnp.exp(vec * INV_SQRT_D)

            def vbody(ii, carry):
                for j in range(4):
                    e = ii * 4 + j
                    w = wb[p, pl.ds(e, 16)][0]
                    for u in range(4):
                        x = eb[p, e, pl.ds(u * 16, 16)]
                        ea = plsc.bitcast(x << 16, jnp.float32)
                        eb2 = plsc.bitcast(x & jnp.int32(-65536), jnp.float32)
                        xv = kvb[p, e, pl.ds(D // 2 + u * 16, 16)]
                        va = plsc.bitcast(xv << 16, jnp.float32)
                        vb2 = plsc.bitcast(xv & jnp.int32(-65536), jnp.float32)
                        sa = pl.ds(u * 32, 16)
                        sb = pl.ds(u * 32 + 16, 16)
                        wvb[p, e, sa] = w * (va + ea)
                        wvb[p, e, sb] = w * (vb2 + eb2)
                return carry

            lax.fori_loop(0, CH // 4, vbody, 0)

        fire(0, 0)

        def pair(g, carry):
            i0 = 2 * g

            @pl.when(g > 0)
            def _():
                wait_writes(1)
            fire(i0 + 1, 1)
            wait_gathers(0)

            @pl.when(g > 0)
            def _():
                wait_writes(0)
            compute(0)
            fire_writes(i0, 0)

            @pl.when(g < N_PAIR - 1)
            def _():
                fire(i0 + 2, 0)
            wait_gathers(1)
            compute(1)
            fire_writes(i0 + 1, 1)
            return carry

        lax.fori_loop(0, N_PAIR, pair, 0)
        wait_writes(0)
        wait_writes(1)

    return edge_k


_edge_pass = _make_edge_pass()


def _make_scatter_kernel():
    mesh = plsc.VectorSubcoreMesh(core_axis_name="c", subcore_axis_name="s")
    rows_per_tile = N_ACC // NS

    @functools.partial(
        pl.kernel,
        mesh=mesh,
        compiler_params=pltpu.CompilerParams(needs_layout_passes=False),
        out_type=[
            jax.ShapeDtypeStruct((NC, N_ACC, D), jnp.float32),
            jax.ShapeDtypeStruct((NW, N_ACC), jnp.float32),
        ],
        scratch_types=[
            pltpu.VMEM((N_CH, CH), jnp.int32),
            pltpu.VMEM((2, CH, D), jnp.float32),
            pltpu.VMEM((2, CH), jnp.float32),
            pltpu.VMEM((N_ACC,), jnp.float32),
            pltpu.VMEM_SHARED((N_ACC, D), jnp.float32),
            pltpu.SemaphoreType.DMA,
            pltpu.SemaphoreType.DMA,
        ],
    )
    def scatter_k(wv_h, w_h, dst2_h, z128_h, zden_h,
                  acc128_o, den_o,
                  dstb, wvb, wb, den_v, acc_sh, isem0, isem1):
        cid = lax.axis_index("c")
        sid = lax.axis_index("s")
        wid = sid * NC + cid
        wbase = wid * E_PER_W
        r0 = sid * rows_per_tile
        isems = (isem0, isem1)

        pltpu.sync_copy(z128_h.at[pl.ds(r0, rows_per_tile)],
                        acc_sh.at[pl.ds(r0, rows_per_tile)])
        pltpu.sync_copy(zden_h, den_v)
        pltpu.sync_copy(dst2_h.at[pl.ds(wid * N_CH, N_CH)], dstb)
        plsc.subcore_barrier()

        def fire(i, p):
            base = wbase + i * CH
            pltpu.async_copy(wv_h.at[pl.ds(base, CH)], wvb.at[p], isems[p])
            pltpu.async_copy(w_h.at[pl.ds(base, CH)], wb.at[p], isems[p])

        def wait_in(p):
            pltpu.make_async_copy(wv_h.at[pl.ds(wbase, CH)], wvb.at[p],
                                  isems[p]).wait()
            pltpu.make_async_copy(w_h.at[pl.ds(wbase, CH)], wb.at[p],
                                  isems[p]).wait()

        def consume(i, p):
            pltpu.sync_copy(wvb.at[p], acc_sh.at[dstb.at[i]], add=True)
            for g in range(CH // 16):
                s = pl.ds(g * 16, 16)
                plsc.addupdate_scatter(den_v, [dstb[i, s]], wb[p, s])

        fire(0, 0)

        def pair(g, carry):
            i0 = 2 * g
            fire(i0 + 1, 1)
            wait_in(0)
            consume(i0, 0)

            @pl.when(g < N_PAIR - 1)
            def _():
                fire(i0 + 2, 0)
            wait_in(1)
            consume(i0 + 1, 1)
            return carry

        lax.fori_loop(0, N_PAIR, pair, 0)
        plsc.subcore_barrier()

        pltpu.sync_copy(acc_sh.at[pl.ds(r0, rows_per_tile)],
                        acc128_o.at[cid, pl.ds(r0, rows_per_tile)])
        pltpu.sync_copy(den_v, den_o.at[wid])

    return scatter_k


_scatter = _make_scatter_kernel()


def _final_kernel(a, dp, z, zskip, out):
    denom = jnp.sum(dp[...], axis=1, keepdims=True) + 1e-16
    agg = a[0] + a[1]
    c = agg / denom + zskip[...]
    zb = z[...]
    out[...] = zb + EPSILON * jnp.tanh(c - GAMMA * zb)


def _final(acc128, den_p, z, zskip):
    blk = 1000
    return pl.pallas_call(
        _final_kernel,
        grid=(N_NODES // blk,),
        in_specs=[
            pl.BlockSpec((NC, blk, D), lambda i: (0, i, 0)),
            pl.BlockSpec((blk, NW), lambda i: (i, 0)),
            pl.BlockSpec((blk, D), lambda i: (i, 0)),
            pl.BlockSpec((blk, D), lambda i: (i, 0)),
        ],
        out_specs=pl.BlockSpec((blk, D), lambda i: (i, 0)),
        out_shape=jax.ShapeDtypeStruct((N_NODES, D), jnp.float32),
    )(acc128[:, :N_NODES], den_p.T[:N_NODES], z, zskip)


def kernel(x, n_id, msg, t, edge_index, memory, last_update, time_w, time_b,
           Wq, bq, Wk, bk, Wv, bv, We, Ws, bs):
    z = memory
    lu = last_update

    src = edge_index[0]
    dst = edge_index[1]
    pad = E_PAD - N_EDGES
    spread = jnp.arange(pad, dtype=jnp.int32)
    src_p = jnp.concatenate([src, spread % N_NODES])
    dstg_p = jnp.concatenate([dst, spread % N_NODES])
    dsts_p = jnp.concatenate([dst, TRASH + spread % (N_ACC - N_NODES)])
    t_p = jnp.concatenate([t, jnp.zeros((pad,), jnp.float32)])
    msg_p = jnp.concatenate([msg, jnp.zeros((pad, 16), jnp.float32)])

    lug = _lu_gather(lu, src_p)

    u4 = jnp.arange(4, dtype=jnp.int32)[:, None] * 32
    l16 = jnp.arange(16, dtype=jnp.int32)[None, :]
    lo_order = (u4 + l16).reshape(-1)
    order = jnp.concatenate([lo_order, lo_order + 16])

    q, kv, zskip = _node_proj(
        z, Wq.T, Wk.T[:, order], Wv.T[:, order], Ws.T,
        bq.reshape(1, D), bk[order].reshape(1, D), bv[order].reshape(1, D),
        bs.reshape(1, D))
    E = _edge_feat(lug.reshape(E_PAD, 1), t_p.reshape(E_PAD, 1), msg_p,
                   time_w.reshape(1, TIME_DIM), time_b.reshape(1, TIME_DIM),
                   We.T[:, order])

    wv, w = _edge_pass(q, kv, E, src_p, dstg_p)

    zeros128 = jnp.zeros((N_ACC, D), jnp.float32)
    zerosden = jnp.zeros((N_ACC,), jnp.float32)
    acc128, den_p = _scatter(wv, w, dsts_p.reshape(NW * N_CH, CH),
                             zeros128, zerosden)

    return _final(acc128, den_p, z, zskip)
